# baseline (speedup 1.0000x reference)
import functools

import jax
import jax.numpy as jnp
from jax import lax
from jax.experimental import pallas as pl
from jax.experimental.pallas import tpu as pltpu
from jax.experimental.pallas import tpu_sc as plsc

SEQ_LEN = 512
CLS_ID = 101
SEP_ID = 102
PAD_ID = 0
B = 16
TOTAL = 4096
BUDGET = SEQ_LEN - 3
SPAN = SEQ_LEN
LANES = 16
STEPS = SPAN // LANES
WIN1 = 640
WIN2 = 1152

_mesh = plsc.VectorSubcoreMesh(core_axis_name="c", subcore_axis_name="s", num_cores=1)


@functools.partial(
    pl.kernel,
    mesh=_mesh,
    compiler_params=pltpu.CompilerParams(needs_layout_passes=False),
    out_type=[
        jax.ShapeDtypeStruct((B, SEQ_LEN), jnp.int32),
        jax.ShapeDtypeStruct((B, SEQ_LEN), jnp.int32),
        jax.ShapeDtypeStruct((B, SEQ_LEN), jnp.int32),
    ],
    scratch_types=[
        pltpu.VMEM((128,), jnp.int32),
        pltpu.VMEM((128,), jnp.int32),
        pltpu.VMEM((WIN1 + WIN2,), jnp.int32),
        pltpu.VMEM((SPAN,), jnp.int32),
        pltpu.VMEM((SPAN,), jnp.int32),
        pltpu.VMEM((SPAN,), jnp.int32),
        pltpu.SemaphoreType.DMA,
        pltpu.SemaphoreType.DMA,
        pltpu.SemaphoreType.DMA,
    ],
)
def _pack_kernel(tok1_hbm, cu1_hbm, tok2_hbm, cu2_hbm,
                 ids_hbm, mask_hbm, type_hbm,
                 cu1_v, cu2_v, win_v, ids_v, mask_v, type_v,
                 sem_cu, sem_tok, sem_out):
    b = lax.axis_index("s")

    hc1 = pltpu.async_copy(cu1_hbm, cu1_v.at[pl.ds(0, B + 1)], sem_cu)
    hc2 = pltpu.async_copy(cu2_hbm, cu2_v.at[pl.ds(0, B + 1)], sem_cu)
    hc1.wait()
    hc2.wait()

    bv = jnp.full((LANES,), b, jnp.int32)
    c1lo = plsc.load_gather(cu1_v, [bv])
    c1hi = plsc.load_gather(cu1_v, [bv + 1])
    c2lo = plsc.load_gather(cu2_v, [bv])
    c2hi = plsc.load_gather(cu2_v, [bv + 1])

    raw1 = jnp.max(c1lo) - 1
    raw2 = jnp.max(c2lo) - (BUDGET + 2)
    s1 = pl.multiple_of(jnp.minimum(jnp.maximum(raw1, 0) & ~7, TOTAL - WIN1), 8)
    s2 = pl.multiple_of(jnp.minimum(jnp.maximum(raw2, 0) & ~7, TOTAL - WIN2), 8)
    hw1 = pltpu.async_copy(tok1_hbm.at[pl.ds(s1, WIN1)],
                           win_v.at[pl.ds(0, WIN1)], sem_tok)
    hw2 = pltpu.async_copy(tok2_hbm.at[pl.ds(s2, WIN2)],
                           win_v.at[pl.ds(WIN1, WIN2)], sem_tok)

    len1 = c1hi - c1lo
    len2 = c2hi - c2lo
    t1 = jnp.minimum(len1, BUDGET - jnp.minimum(len2, BUDGET // 2))
    t2 = jnp.minimum(len2, BUDGET - t1)
    tsum2 = t1 + t2 + 2

    lane = lax.iota(jnp.int32, LANES)

    def _mask_body(j, _):
        for u in range(2):
            p = lane + (j * 2 + u) * LANES
            mask = (p <= tsum2).astype(jnp.int32)
            tids = ((p >= t1 + 2) & (p <= tsum2)).astype(jnp.int32)
            sl = pl.ds((j * 2 + u) * LANES, LANES)
            mask_v[sl] = mask
            type_v[sl] = tids
        return 0

    lax.fori_loop(0, STEPS // 2, _mask_body, 0)
    out_sl = pl.ds(0, SPAN)
    hm = pltpu.async_copy(mask_v, mask_hbm.at[b, out_sl], sem_out)
    htp = pltpu.async_copy(type_v, type_hbm.at[b, out_sl], sem_out)

    hw1.wait()
    hw2.wait()

    s1v = jnp.full((LANES,), s1, jnp.int32)
    s2v = jnp.full((LANES,), s2, jnp.int32)

    def _ids_body(j, _):
        for u in range(2):
            p = lane + (j * 2 + u) * LANES
            in1 = (p >= 1) & (p <= t1)
            in2 = (p >= t1 + 2) & (p <= tsum2 - 1)
            sep = (p == t1 + 1) | (p == tsum2)
            idx1 = jnp.clip(c1lo + p - 1, 0, TOTAL - 1) - s1v
            idx2 = jnp.clip(c2lo + p - t1 - 2, 0, TOTAL - 1) - s2v + WIN1
            g = plsc.load_gather(win_v, [jnp.where(in1, idx1, idx2)])
            ids = jnp.where(p == 0, CLS_ID,
                  jnp.where(in1 | in2, g,
                  jnp.where(sep, SEP_ID, PAD_ID))).astype(jnp.int32)
            ids_v[pl.ds((j * 2 + u) * LANES, LANES)] = ids
        return 0

    lax.fori_loop(0, STEPS // 2, _ids_body, 0)

    hi = pltpu.async_copy(ids_v, ids_hbm.at[b, out_sl], sem_out)
    hm.wait()
    htp.wait()
    hi.wait()


def kernel(tokens1, cu_seqlens1, tokens2, cu_seqlens2, label):
    ids, mask, type_ids = _pack_kernel(tokens1, cu_seqlens1, tokens2, cu_seqlens2)
    return (ids, mask, type_ids, label)


# speedup vs baseline: 1.0120x; 1.0120x over previous
import functools

import jax
import jax.numpy as jnp
from jax import lax
from jax.experimental import pallas as pl
from jax.experimental.pallas import tpu as pltpu
from jax.experimental.pallas import tpu_sc as plsc

SEQ_LEN = 512
CLS_ID = 101
SEP_ID = 102
PAD_ID = 0
B = 16
TOTAL = 4096
BUDGET = SEQ_LEN - 3
SPAN = SEQ_LEN
LANES = 16
STEPS = SPAN // LANES
WIN1 = 640
WIN2 = 1152

_mesh = plsc.VectorSubcoreMesh(core_axis_name="c", subcore_axis_name="s", num_cores=1)


@functools.partial(
    pl.kernel,
    mesh=_mesh,
    compiler_params=pltpu.CompilerParams(needs_layout_passes=False),
    out_type=[
        jax.ShapeDtypeStruct((B, SEQ_LEN), jnp.int32),
        jax.ShapeDtypeStruct((B, SEQ_LEN), jnp.int32),
        jax.ShapeDtypeStruct((B, SEQ_LEN), jnp.int32),
    ],
    scratch_types=[
        pltpu.VMEM((128,), jnp.int32),
        pltpu.VMEM((128,), jnp.int32),
        pltpu.VMEM((WIN1 + WIN2,), jnp.int32),
        pltpu.VMEM((SPAN,), jnp.int32),
        pltpu.VMEM((SPAN,), jnp.int32),
        pltpu.VMEM((SPAN,), jnp.int32),
        pltpu.SemaphoreType.DMA,
        pltpu.SemaphoreType.DMA,
        pltpu.SemaphoreType.DMA,
    ],
)
def _pack_kernel(tok1_hbm, cu1_hbm, tok2_hbm, cu2_hbm,
                 ids_hbm, mask_hbm, type_hbm,
                 cu1_v, cu2_v, win_v, ids_v, mask_v, type_v,
                 sem_cu, sem_tok, sem_out):
    b = lax.axis_index("s")

    hc1 = pltpu.async_copy(cu1_hbm, cu1_v.at[pl.ds(0, B + 1)], sem_cu)
    hc2 = pltpu.async_copy(cu2_hbm, cu2_v.at[pl.ds(0, B + 1)], sem_cu)
    hc1.wait()
    hc2.wait()

    bv = jnp.full((LANES,), b, jnp.int32)
    c1lo = plsc.load_gather(cu1_v, [bv])
    c1hi = plsc.load_gather(cu1_v, [bv + 1])
    c2lo = plsc.load_gather(cu2_v, [bv])
    c2hi = plsc.load_gather(cu2_v, [bv + 1])

    raw1 = jnp.max(c1lo) - 1
    raw2 = jnp.max(c2lo) - (BUDGET + 2)
    s1 = pl.multiple_of(jnp.minimum(jnp.maximum(raw1, 0) & ~7, TOTAL - WIN1), 8)
    s2 = pl.multiple_of(jnp.minimum(jnp.maximum(raw2, 0) & ~7, TOTAL - WIN2), 8)
    hw1 = pltpu.async_copy(tok1_hbm.at[pl.ds(s1, WIN1)],
                           win_v.at[pl.ds(0, WIN1)], sem_tok)
    hw2 = pltpu.async_copy(tok2_hbm.at[pl.ds(s2, WIN2)],
                           win_v.at[pl.ds(WIN1, WIN2)], sem_tok)

    len1 = c1hi - c1lo
    len2 = c2hi - c2lo
    t1 = jnp.minimum(len1, BUDGET - jnp.minimum(len2, BUDGET // 2))
    t2 = jnp.minimum(len2, BUDGET - t1)
    tsum2 = t1 + t2 + 2

    lane = lax.iota(jnp.int32, LANES)

    def _mask_body(j, _):
        p = lane + j * LANES
        mask = (p <= tsum2).astype(jnp.int32)
        tids = ((p >= t1 + 2) & (p <= tsum2)).astype(jnp.int32)
        sl = pl.ds(j * LANES, LANES)
        mask_v[sl] = mask
        type_v[sl] = tids
        return 0

    lax.fori_loop(0, STEPS, _mask_body, 0)
    out_sl = pl.ds(0, SPAN)
    hm = pltpu.async_copy(mask_v, mask_hbm.at[b, out_sl], sem_out)
    htp = pltpu.async_copy(type_v, type_hbm.at[b, out_sl], sem_out)

    hw1.wait()
    hw2.wait()

    s1v = jnp.full((LANES,), s1, jnp.int32)
    s2v = jnp.full((LANES,), s2, jnp.int32)

    def _ids_body(j, _):
        p = lane + j * LANES
        in1 = (p >= 1) & (p <= t1)
        in2 = (p >= t1 + 2) & (p <= tsum2 - 1)
        sep = (p == t1 + 1) | (p == tsum2)
        idx1 = jnp.clip(c1lo + p - 1, 0, TOTAL - 1) - s1v
        idx2 = jnp.clip(c2lo + p - t1 - 2, 0, TOTAL - 1) - s2v + WIN1
        g = plsc.load_gather(win_v, [jnp.where(in1, idx1, idx2)])
        ids = jnp.where(p == 0, CLS_ID,
              jnp.where(in1 | in2, g,
              jnp.where(sep, SEP_ID, PAD_ID))).astype(jnp.int32)
        ids_v[pl.ds(j * LANES, LANES)] = ids
        return 0

    lax.fori_loop(0, STEPS, _ids_body, 0)

    hi = pltpu.async_copy(ids_v, ids_hbm.at[b, out_sl], sem_out)
    hm.wait()
    htp.wait()
    hi.wait()


def kernel(tokens1, cu_seqlens1, tokens2, cu_seqlens2, label):
    ids, mask, type_ids = _pack_kernel(tokens1, cu_seqlens1, tokens2, cu_seqlens2)
    return (ids, mask, type_ids, label)
